# baseline (speedup 1.0000x reference)
import jax
import jax.numpy as jnp
from jax.experimental import pallas as pl

NUM_CLASSES = 1000
N = 16384
M = 26
W = 32
IB = 8192


def _onehot_body(idx_ref, out_ref):
    idx = idx_ref[...][0]
    word_idx = idx >> 5
    val = jnp.left_shift(jnp.int32(1), idx & 31)
    w_iota = jax.lax.broadcasted_iota(jnp.int32, (1, W, IB), 1)
    out_ref[...] = jnp.where(word_idx[:, None, :] == w_iota, val[:, None, :], 0)


def kernel(index):
    idx_t = index.T.reshape(M, 1, N)
    packed = pl.pallas_call(
        _onehot_body,
        grid=(M, N // IB),
        in_specs=[pl.BlockSpec((1, 1, IB), lambda j, i: (j, 0, i))],
        out_specs=pl.BlockSpec((1, W, IB), lambda j, i: (j, 0, i)),
        out_shape=jax.ShapeDtypeStruct((M, W, N), jnp.int32),
    )(idx_t)
    bits = jax.lax.broadcasted_iota(jnp.int32, (1, 1, 32, 1), 2)
    oh_t = ((packed[:, :, None, :] >> bits) & 1).astype(jnp.bool_)
    oh_t = oh_t.reshape(M, 32 * W, N)[:, :NUM_CLASSES, :]
    return oh_t.transpose(2, 0, 1)


# speedup vs baseline: 2.5473x; 2.5473x over previous
import jax
import jax.numpy as jnp
from jax.experimental import pallas as pl

NUM_CLASSES = 1000
N = 16384
M = 26
IB = 8192


def _onehot_body(idx_ref, out_ref):
    idx = idx_ref[...][0]
    word_idx = idx >> 2
    val = jnp.left_shift(jnp.int32(1), 8 * (idx & 3))
    w_iota = jax.lax.broadcasted_iota(jnp.int32, (1, NUM_CLASSES // 4, IB), 1)
    words = jnp.where(word_idx[:, None, :] == w_iota, val[:, None, :], 0)
    out_ref.bitcast(jnp.int32)[...] = words


def kernel(index):
    idx_t = index.T.reshape(M, 1, N)
    oh_t = pl.pallas_call(
        _onehot_body,
        grid=(M, N // IB),
        in_specs=[pl.BlockSpec((1, 1, IB), lambda j, i: (j, 0, i))],
        out_specs=pl.BlockSpec((1, NUM_CLASSES, IB), lambda j, i: (j, 0, i)),
        out_shape=jax.ShapeDtypeStruct((M, NUM_CLASSES, N), jnp.int8),
    )(idx_t)
    return oh_t.transpose(2, 0, 1).astype(jnp.bool_)
